# baseline (speedup 1.0000x reference)
import functools

import jax
import jax.numpy as jnp
from jax import lax
from jax.experimental import pallas as pl
from jax.experimental.pallas import tpu as pltpu
from jax.experimental.pallas import tpu_sc as plsc

B, Q, T = 64, 300, 30
QP, TP = 320, 32
L = 16
NCH = QP // L
NCHS = 19
NW = 32
SPW = B // NW
BIG = 1e30


def _prep_body(x_ref, tgt_ref, ew_ref, lp_ref):
    x = x_ref[0]
    m = jnp.max(x, axis=-1, keepdims=True)
    e = jnp.exp(x - m)
    s = jnp.sum(e, axis=-1, keepdims=True)
    lse = m + jnp.log(s)
    tg = tgt_ref[0, 0, :]
    ncls = x.shape[-1]
    cls = lax.broadcasted_iota(jnp.int32, (T, ncls), 1)
    onehot = (cls == tg[:, None]).astype(jnp.float32)
    dn = (((1,), (1,)), ((), ()))
    g = lax.dot_general(onehot, x, dn,
                        preferred_element_type=jnp.float32)
    lp_t = g - lse[:, 0][None, :]
    lp_last = x[:, ncls - 1] - lse[:, 0]
    ew = ew_ref[0]
    w = lax.dot_general(onehot, ew, (((1,), (0,)), ((), ())))
    ew_c = ew[ncls - 1]

    lp_ref[0] = jnp.zeros((TP, QP), jnp.float32)
    lp_ref[0, 0:T, 0:Q] = lp_t
    lp_ref[0, T:T + 1, 0:Q] = lp_last[None, :]
    lp_ref[0, TP - 1:TP, 0:T] = w[None, :]
    lp_ref[0, TP - 1:TP, T:T + 1] = ew_c[None, None]


def _prep(outputs, targets, empty_weight):
    tgt3 = targets.astype(jnp.int32).reshape(B, 1, T)
    ew2 = empty_weight.reshape(1, -1)
    return pl.pallas_call(
        _prep_body,
        grid=(B,),
        in_specs=[
            pl.BlockSpec((1, Q, outputs.shape[-1]), lambda b: (b, 0, 0)),
            pl.BlockSpec((1, 1, T), lambda b: (b, 0, 0)),
            pl.BlockSpec((1, empty_weight.shape[0]), lambda b: (0, 0)),
        ],
        out_specs=pl.BlockSpec((1, TP, QP), lambda b: (b, 0, 0)),
        out_shape=jax.ShapeDtypeStruct((B, TP, QP), jnp.float32),
    )(outputs, tgt3, ew2)


def _sc_body(lp_hbm, out_hbm,
             lp_a, lp_b, v_v, sh_v, path_v, sc_v, r4c_v, out_v, red_v, redi_v,
             c4r_s, u_s, sl_s, cl_s, st_i, st_f, sem_a, sem_b):
    wid = lax.axis_index("s") * 2 + lax.axis_index("c")
    zeros16 = jnp.zeros((L,), jnp.float32)
    iota16 = lax.iota(jnp.int32, L)
    padmask = iota16 >= (Q - (NCHS - 1) * L)

    cp_a = pltpu.async_copy(lp_hbm.at[wid * SPW], lp_a, sem_a)
    cp_b = pltpu.async_copy(lp_hbm.at[wid * SPW + 1], lp_b, sem_b)

    def tree_min_f32(vec):
        r = vec
        for off in (8, 4, 2, 1):
            red_v[pl.ds(0, L)] = r
            r = jnp.minimum(r, red_v[pl.ds(off, L)])
        return r[0]

    def tree_min_i32(vec):
        r = vec
        for off in (8, 4, 2, 1):
            redi_v[pl.ds(0, L)] = r
            r = jnp.minimum(r, redi_v[pl.ds(off, L)])
        return r[0]

    red_v[pl.ds(L, L)] = jnp.full((L,), BIG, jnp.float32)
    redi_v[pl.ds(L, L)] = jnp.full((L,), 2 ** 30, jnp.int32)

    def pick_scan(minv, mini):
        mval = tree_min_f32(minv)
        jloc = tree_min_i32(jnp.where(minv == mval, mini, jnp.int32(2 ** 30)))
        return mval, jloc

    out_vec = zeros16
    for sidx, (lp_v, cp) in enumerate(((lp_a, cp_a), (lp_b, cp_b))):
        cp.wait()

        def init_vec(k, c):
            v_v[pl.ds(k * L, L)] = zeros16
            r4c_v[pl.ds(k * L, L)] = jnp.full((L,), -1, jnp.int32)
            return c
        lax.fori_loop(0, NCH, init_vec, 0)

        def init_rows(r, c):
            u_s[r] = 0.0
            c4r_s[r] = -1
            return c
        lax.fori_loop(0, T, init_rows, 0)

        def outer(cur_row, carry):
            minv = jnp.full((L,), BIG, jnp.float32)
            mini = jnp.zeros((L,), jnp.int32)
            base = cur_row * QP
            for k in range(NCHS):
                sl = pl.ds(k * L, L)
                d = -jnp.exp(lp_v[pl.ds(base + k * L, L)]) - v_v[sl]
                if k == NCHS - 1:
                    d = jnp.where(padmask, BIG, d)
                sh_v[sl] = d
                path_v[sl] = jnp.zeros((L,), jnp.int32) + cur_row
                sc_v[sl] = zeros16
                cnd = d < minv
                minv = jnp.where(cnd, d, minv)
                mini = jnp.where(cnd, iota16 + k * L, mini)
            mval, jloc = pick_scan(minv, mini)
            scj = sc_v[pl.ds(jloc, L)]
            sc_v[pl.ds(jloc, L)] = jnp.where(iota16 == 0, 1.0, scj)
            rj = r4c_v[pl.ds(jloc, L)][0]
            free = rj == -1
            sl_s[0] = cur_row
            cl_s[0] = jloc
            st_i[4] = 1
            st_i[0] = jnp.where(free, cur_row, rj)
            st_i[1] = jnp.where(free, jloc, jnp.int32(-1))
            st_f[0] = mval

            def sp_step(it, c):
                @pl.when(st_i[1] == -1)
                def _():
                    i = st_i[0]
                    min_val = st_f[0]
                    cnt = st_i[4]
                    sl_s[cnt] = i
                    ui = u_s[i]
                    ibase = i * QP
                    minv = jnp.full((L,), BIG, jnp.float32)
                    mini = jnp.zeros((L,), jnp.int32)
                    for k in range(NCHS):
                        sl = pl.ds(k * L, L)
                        d = (min_val
                             - jnp.exp(lp_v[pl.ds(ibase + k * L, L)])
                             - ui - v_v[sl])
                        if k == NCHS - 1:
                            d = jnp.where(padmask, BIG, d)
                        sck = sc_v[sl]
                        shk = sh_v[sl]
                        bet = (sck == 0.0) & (d < shk)
                        shk = jnp.where(bet, d, shk)
                        sh_v[sl] = shk
                        path_v[sl] = jnp.where(bet, i, path_v[sl])
                        masked = jnp.where(sck == 0.0, shk, BIG)
                        cnd = masked < minv
                        minv = jnp.where(cnd, masked, minv)
                        mini = jnp.where(cnd, iota16 + k * L, mini)
                    mval, jloc = pick_scan(minv, mini)
                    scj = sc_v[pl.ds(jloc, L)]
                    sc_v[pl.ds(jloc, L)] = jnp.where(iota16 == 0, 1.0, scj)
                    cl_s[cnt] = jloc
                    st_i[4] = cnt + 1
                    rj = r4c_v[pl.ds(jloc, L)][0]
                    free = rj == -1
                    st_i[1] = jnp.where(free, jloc, jnp.int32(-1))
                    st_i[0] = jnp.where(free, i, rj)
                    st_f[0] = mval
                return c

            lax.fori_loop(0, cur_row, sp_step, 0)
            min_val = st_f[0]
            sink = st_i[1]
            cnt = st_i[4]

            u_s[cur_row] = u_s[cur_row] + min_val

            def u_upd(idx, c):
                r = sl_s[idx]
                jr = c4r_s[r]
                u_s[r] = u_s[r] + (min_val - sh_v[pl.ds(jr, L)][0])
                return c
            lax.fori_loop(1, cnt, u_upd, 0)

            def v_upd(idx, c):
                j = cl_s[idx]
                sh16 = sh_v[pl.ds(j, L)]
                vv16 = v_v[pl.ds(j, L)]
                v_v[pl.ds(j, L)] = jnp.where(iota16 == 0,
                                             vv16 - (min_val - sh16), vv16)
                return c
            lax.fori_loop(0, cnt, v_upd, 0)

            st_i[2] = 0
            st_i[3] = sink

            def aug_step(it, c):
                @pl.when(st_i[2] == 0)
                def _():
                    j = st_i[3]
                    i = path_v[pl.ds(j, L)][0]
                    rv16 = r4c_v[pl.ds(j, L)]
                    r4c_v[pl.ds(j, L)] = jnp.where(iota16 == 0, i, rv16)
                    nj = c4r_s[i]
                    c4r_s[i] = j
                    st_i[3] = nj
                    st_i[2] = jnp.where(i == cur_row, 1, 0)
                return c

            lax.fori_loop(0, cnt, aug_step, 0)
            return carry

        lax.fori_loop(0, T, outer, 0)

        def s2_chunk(k, acc):
            return acc + lp_v[pl.ds(T * QP + k * L, L)]
        s2v = lax.fori_loop(0, NCH, s2_chunk, zeros16)
        red_v[pl.ds(L, L)] = zeros16
        s2 = s2v
        for off in (8, 4, 2, 1):
            red_v[pl.ds(0, L)] = s2
            s2 = s2 + red_v[pl.ds(off, L)]
        s2 = s2[0]
        red_v[pl.ds(L, L)] = jnp.full((L,), BIG, jnp.float32)

        def t_loop(t, acc):
            a1, a3, ws = acc
            c = c4r_s[t]
            wt = lp_v[pl.ds((TP - 1) * QP + t, L)][0]
            return (a1 + wt * lp_v[pl.ds(t * QP + c, L)][0],
                    a3 + lp_v[pl.ds(T * QP + c, L)][0],
                    ws + wt)
        a1, a3, wsum = lax.fori_loop(
            0, T, t_loop,
            (jnp.float32(0.0), jnp.float32(0.0), jnp.float32(0.0)))
        ew_c = lp_v[pl.ds((TP - 1) * QP + T, L)][0]
        numer = a1 + ew_c * (s2 - a3)
        denom = wsum + (Q - T) * ew_c
        out_vec = (out_vec
                   + jnp.where(iota16 == 2 * sidx, numer, 0.0)
                   + jnp.where(iota16 == 2 * sidx + 1, denom, 0.0))

    out_v[...] = out_vec
    pltpu.sync_copy(out_v, out_hbm.at[wid])


@functools.partial(jax.jit, static_argnames=())
def _match_loss(lp_all):
    mesh = plsc.VectorSubcoreMesh(core_axis_name="c", subcore_axis_name="s")
    lp_all = lp_all.reshape(B, TP * QP)
    f = pl.kernel(
        _sc_body,
        out_type=jax.ShapeDtypeStruct((NW, L), jnp.float32),
        mesh=mesh,
        scratch_types=[
            pltpu.VMEM((TP * QP,), jnp.float32),
            pltpu.VMEM((TP * QP,), jnp.float32),
            pltpu.VMEM((QP,), jnp.float32),
            pltpu.VMEM((QP,), jnp.float32),
            pltpu.VMEM((QP,), jnp.int32),
            pltpu.VMEM((QP,), jnp.float32),
            pltpu.VMEM((QP,), jnp.int32),
            pltpu.VMEM((L,), jnp.float32),
            pltpu.VMEM((2 * L,), jnp.float32),
            pltpu.VMEM((2 * L,), jnp.int32),
            pltpu.SMEM((TP,), jnp.int32),
            pltpu.SMEM((TP,), jnp.float32),
            pltpu.SMEM((TP,), jnp.int32),
            pltpu.SMEM((TP,), jnp.int32),
            pltpu.SMEM((8,), jnp.int32),
            pltpu.SMEM((8,), jnp.float32),
            pltpu.SemaphoreType.DMA,
            pltpu.SemaphoreType.DMA,
        ],
    )
    return f(lp_all)


def kernel(outputs, targets, empty_weight):
    lp_all = _prep(outputs, targets, empty_weight)
    part = _match_loss(lp_all)
    numer = part[:, 0::2].sum()
    denom = part[:, 1::2].sum()
    return -(numer / denom)


# speedup vs baseline: 1.0476x; 1.0476x over previous
import functools

import jax
import jax.numpy as jnp
from jax import lax
from jax.experimental import pallas as pl
from jax.experimental.pallas import tpu as pltpu
from jax.experimental.pallas import tpu_sc as plsc

B, Q, T = 64, 300, 30
QP, TP = 320, 32
L = 16
NCH = QP // L
NCHS = 19
NW = 32
SPW = B // NW
BIG = 1e30


def _prep_body(x_ref, tgt_ref, ew_ref, cost_ref, lp_ref):
    x = x_ref[0]
    m = jnp.max(x, axis=-1, keepdims=True)
    e = jnp.exp(x - m)
    s = jnp.sum(e, axis=-1, keepdims=True)
    lse = m + jnp.log(s)
    tg = tgt_ref[0, 0, :]
    ncls = x.shape[-1]
    cls = lax.broadcasted_iota(jnp.int32, (T, ncls), 1)
    onehot = (cls == tg[:, None]).astype(jnp.float32)
    dn = (((1,), (1,)), ((), ()))
    g = lax.dot_general(onehot, x, dn,
                        preferred_element_type=jnp.float32)
    lp_t = g - lse[:, 0][None, :]
    cost = -jnp.exp(lp_t)
    lp_last = x[:, ncls - 1] - lse[:, 0]
    ew = ew_ref[0]
    w = lax.dot_general(onehot, ew, (((1,), (0,)), ((), ())))
    ew_c = ew[ncls - 1]

    cost_ref[0] = jnp.full((TP, QP), BIG, jnp.float32)
    cost_ref[0, 0:T, 0:Q] = cost
    cost_ref[0, TP - 1:TP, 0:T] = w[None, :]
    cost_ref[0, TP - 1:TP, T:T + 1] = ew_c[None, None]
    lp_ref[0] = jnp.zeros((TP, QP), jnp.float32)
    lp_ref[0, 0:T, 0:Q] = lp_t
    lp_ref[0, T:T + 1, 0:Q] = lp_last[None, :]


def _prep(outputs, targets, empty_weight):
    tgt3 = targets.astype(jnp.int32).reshape(B, 1, T)
    ew2 = empty_weight.reshape(1, -1)
    return pl.pallas_call(
        _prep_body,
        grid=(B,),
        in_specs=[
            pl.BlockSpec((1, Q, outputs.shape[-1]), lambda b: (b, 0, 0)),
            pl.BlockSpec((1, 1, T), lambda b: (b, 0, 0)),
            pl.BlockSpec((1, empty_weight.shape[0]), lambda b: (0, 0)),
        ],
        out_specs=[
            pl.BlockSpec((1, TP, QP), lambda b: (b, 0, 0)),
            pl.BlockSpec((1, TP, QP), lambda b: (b, 0, 0)),
        ],
        out_shape=[
            jax.ShapeDtypeStruct((B, TP, QP), jnp.float32),
            jax.ShapeDtypeStruct((B, TP, QP), jnp.float32),
        ],
    )(outputs, tgt3, ew2)


def _sc_body(cost_hbm, lp_hbm, out_hbm,
             cost_a, cost_b, lp_a, lp_b,
             v_v, sh_v, path_v, sc_v, r4c_v, out_v,
             c4r_s, u_s, sl_s, cl_s, st_i, st_f,
             sem_ca, sem_cb, sem_la, sem_lb):
    wid = lax.axis_index("s") * 2 + lax.axis_index("c")
    zeros16 = jnp.zeros((L,), jnp.float32)
    iota16 = lax.iota(jnp.int32, L)

    cp_ca = pltpu.async_copy(cost_hbm.at[wid * SPW], cost_a, sem_ca)
    cp_cb = pltpu.async_copy(cost_hbm.at[wid * SPW + 1], cost_b, sem_cb)
    cp_la = pltpu.async_copy(lp_hbm.at[wid * SPW], lp_a, sem_la)
    cp_lb = pltpu.async_copy(lp_hbm.at[wid * SPW + 1], lp_b, sem_lb)

    def rot(vec, off):
        return vec.at[(iota16 + off) % L].get(mode="promise_in_bounds")

    def shuf_min(vec):
        m = vec
        for off in (8, 4, 2, 1):
            m = jnp.minimum(m, rot(m, off))
        return m

    def shuf_sum(vec):
        m = vec
        for off in (8, 4, 2, 1):
            m = m + rot(m, off)
        return m

    def pick_scan(minv, mini):
        mv = shuf_min(minv)
        cand = jnp.where(minv == mv, mini, jnp.int32(2 ** 30))
        return mv[0], shuf_min(cand)[0]

    out_vec = zeros16
    for sidx, (cost_v, lp_v, cp_c, cp_l) in enumerate(
            ((cost_a, lp_a, cp_ca, cp_la), (cost_b, lp_b, cp_cb, cp_lb))):
        cp_c.wait()

        def init_vec(k, c):
            v_v[pl.ds(k * L, L)] = zeros16
            r4c_v[pl.ds(k * L, L)] = jnp.full((L,), -1, jnp.int32)
            return c
        lax.fori_loop(0, NCH, init_vec, 0)

        def init_rows(r, c):
            u_s[r] = 0.0
            c4r_s[r] = -1
            return c
        lax.fori_loop(0, T, init_rows, 0)

        def outer(cur_row, carry):
            minv = jnp.full((L,), BIG, jnp.float32)
            mini = jnp.zeros((L,), jnp.int32)
            base = cur_row * QP
            for k in range(NCHS):
                sl = pl.ds(k * L, L)
                d = cost_v[pl.ds(base + k * L, L)] - v_v[sl]
                sh_v[sl] = d
                path_v[sl] = jnp.zeros((L,), jnp.int32) + cur_row
                sc_v[sl] = zeros16
                cnd = d < minv
                minv = jnp.where(cnd, d, minv)
                mini = jnp.where(cnd, iota16 + k * L, mini)
            mval, jloc = pick_scan(minv, mini)
            scj = sc_v[pl.ds(jloc, L)]
            sc_v[pl.ds(jloc, L)] = jnp.where(iota16 == 0, 1.0, scj)
            rj = r4c_v[pl.ds(jloc, L)][0]
            free = rj == -1
            sl_s[0] = cur_row
            cl_s[0] = jloc
            st_i[4] = 1
            st_i[0] = jnp.where(free, cur_row, rj)
            st_i[1] = jnp.where(free, jloc, jnp.int32(-1))
            st_f[0] = mval

            def sp_step(it, c):
                @pl.when(st_i[1] == -1)
                def _():
                    i = st_i[0]
                    min_val = st_f[0]
                    cnt = st_i[4]
                    sl_s[cnt] = i
                    ui = u_s[i]
                    ibase = i * QP
                    minv = jnp.full((L,), BIG, jnp.float32)
                    mini = jnp.zeros((L,), jnp.int32)
                    for k in range(NCHS):
                        sl = pl.ds(k * L, L)
                        d = (min_val + cost_v[pl.ds(ibase + k * L, L)]
                             - ui - v_v[sl])
                        sck = sc_v[sl]
                        shk = sh_v[sl]
                        bet = (sck == 0.0) & (d < shk)
                        shk = jnp.where(bet, d, shk)
                        sh_v[sl] = shk
                        path_v[sl] = jnp.where(bet, i, path_v[sl])
                        masked = jnp.where(sck == 0.0, shk, BIG)
                        cnd = masked < minv
                        minv = jnp.where(cnd, masked, minv)
                        mini = jnp.where(cnd, iota16 + k * L, mini)
                    mval, jloc = pick_scan(minv, mini)
                    scj = sc_v[pl.ds(jloc, L)]
                    sc_v[pl.ds(jloc, L)] = jnp.where(iota16 == 0, 1.0, scj)
                    cl_s[cnt] = jloc
                    st_i[4] = cnt + 1
                    rj = r4c_v[pl.ds(jloc, L)][0]
                    free = rj == -1
                    st_i[1] = jnp.where(free, jloc, jnp.int32(-1))
                    st_i[0] = jnp.where(free, i, rj)
                    st_f[0] = mval
                return c

            lax.fori_loop(0, cur_row, sp_step, 0)
            min_val = st_f[0]
            sink = st_i[1]
            cnt = st_i[4]

            u_s[cur_row] = u_s[cur_row] + min_val

            def u_upd(idx, c):
                r = sl_s[idx]
                jr = c4r_s[r]
                u_s[r] = u_s[r] + (min_val - sh_v[pl.ds(jr, L)][0])
                return c
            lax.fori_loop(1, cnt, u_upd, 0)

            def v_upd(idx, c):
                j = cl_s[idx]
                sh16 = sh_v[pl.ds(j, L)]
                vv16 = v_v[pl.ds(j, L)]
                v_v[pl.ds(j, L)] = jnp.where(iota16 == 0,
                                             vv16 - (min_val - sh16), vv16)
                return c
            lax.fori_loop(0, cnt, v_upd, 0)

            st_i[2] = 0
            st_i[3] = sink

            def aug_step(it, c):
                @pl.when(st_i[2] == 0)
                def _():
                    j = st_i[3]
                    i = path_v[pl.ds(j, L)][0]
                    rv16 = r4c_v[pl.ds(j, L)]
                    r4c_v[pl.ds(j, L)] = jnp.where(iota16 == 0, i, rv16)
                    nj = c4r_s[i]
                    c4r_s[i] = j
                    st_i[3] = nj
                    st_i[2] = jnp.where(i == cur_row, 1, 0)
                return c

            lax.fori_loop(0, cnt, aug_step, 0)
            return carry

        lax.fori_loop(0, T, outer, 0)

        cp_l.wait()

        def s2_chunk(k, acc):
            return acc + lp_v[pl.ds(T * QP + k * L, L)]
        s2 = shuf_sum(lax.fori_loop(0, NCH, s2_chunk, zeros16))[0]

        def t_loop(t, acc):
            a1, a3, ws = acc
            c = c4r_s[t]
            wt = cost_v[pl.ds((TP - 1) * QP + t, L)][0]
            return (a1 + wt * lp_v[pl.ds(t * QP + c, L)][0],
                    a3 + lp_v[pl.ds(T * QP + c, L)][0],
                    ws + wt)
        a1, a3, wsum = lax.fori_loop(
            0, T, t_loop,
            (jnp.float32(0.0), jnp.float32(0.0), jnp.float32(0.0)))
        ew_c = cost_v[pl.ds((TP - 1) * QP + T, L)][0]
        numer = a1 + ew_c * (s2 - a3)
        denom = wsum + (Q - T) * ew_c
        out_vec = (out_vec
                   + jnp.where(iota16 == 2 * sidx, numer, 0.0)
                   + jnp.where(iota16 == 2 * sidx + 1, denom, 0.0))

    out_v[...] = out_vec
    pltpu.sync_copy(out_v, out_hbm.at[wid])


@functools.partial(jax.jit, static_argnames=())
def _match_loss(cost_all, lp_all):
    mesh = plsc.VectorSubcoreMesh(core_axis_name="c", subcore_axis_name="s")
    cost_all = cost_all.reshape(B, TP * QP)
    lp_all = lp_all.reshape(B, TP * QP)
    f = pl.kernel(
        _sc_body,
        out_type=jax.ShapeDtypeStruct((NW, L), jnp.float32),
        mesh=mesh,
        scratch_types=[
            pltpu.VMEM((TP * QP,), jnp.float32),
            pltpu.VMEM((TP * QP,), jnp.float32),
            pltpu.VMEM((TP * QP,), jnp.float32),
            pltpu.VMEM((TP * QP,), jnp.float32),
            pltpu.VMEM((QP,), jnp.float32),
            pltpu.VMEM((QP,), jnp.float32),
            pltpu.VMEM((QP,), jnp.int32),
            pltpu.VMEM((QP,), jnp.float32),
            pltpu.VMEM((QP,), jnp.int32),
            pltpu.VMEM((L,), jnp.float32),
            pltpu.SMEM((TP,), jnp.int32),
            pltpu.SMEM((TP,), jnp.float32),
            pltpu.SMEM((TP,), jnp.int32),
            pltpu.SMEM((TP,), jnp.int32),
            pltpu.SMEM((8,), jnp.int32),
            pltpu.SMEM((8,), jnp.float32),
            pltpu.SemaphoreType.DMA,
            pltpu.SemaphoreType.DMA,
            pltpu.SemaphoreType.DMA,
            pltpu.SemaphoreType.DMA,
        ],
    )
    return f(cost_all, lp_all)


def kernel(outputs, targets, empty_weight):
    cost_all, lp_all = _prep(outputs, targets, empty_weight)
    part = _match_loss(cost_all, lp_all)
    numer = part[:, 0::2].sum()
    denom = part[:, 1::2].sum()
    return -(numer / denom)
